# baseline (speedup 1.0000x reference)
import math

import jax
import jax.numpy as jnp
from jax.experimental import pallas as pl
from jax.experimental.pallas import tpu as pltpu

N = 5000
PRED = 85
NCLS = 80
NPAD = 5120
TI = 1280
T = NPAD // TI
TA = 1280
DSTEPS = NPAD // TA
CW = 2
IOU_THR = 0.5

_steps = []
for _t in range(T - 1, -1, -1):
    _nq = math.ceil((_t + 1) / CW)
    for _q in range(_nq):
        _steps.append((_t, _q, 1 if _q == 0 else 0, 1 if _q == _nq - 1 else 0))
SSTEPS = len(_steps)
GRID = DSTEPS + SSTEPS
_tbl = [[0, 0, 0, 0]] * DSTEPS + [list(s) for s in _steps]


def _decode_tile(p, base):
    cx = p[:, 0:1]
    cy = p[:, 1:2]
    w = p[:, 2:3]
    h = p[:, 3:4]
    conf = p[:, 4:5]
    cls = p[:, 5:PRED]
    m = jnp.max(cls, axis=1, keepdims=True)
    iota = jax.lax.broadcasted_iota(jnp.int32, cls.shape, 1)
    amax = jnp.min(jnp.where(cls == m, iota, NCLS), axis=1, keepdims=True)
    x1 = cx - w * 0.5
    y1 = cy - h * 0.5
    x2 = cx + w * 0.5
    y2 = cy + h * 0.5
    score = conf * m
    area = jnp.maximum(x2 - x1, 0.0) * jnp.maximum(y2 - y1, 0.0)
    zero = jnp.zeros_like(score)
    feats = jnp.concatenate(
        [x1, y1, x2, y2, score, amax.astype(jnp.float32), area, zero], axis=1
    )
    gid = base + jax.lax.broadcasted_iota(jnp.int32, (TI, 1), 0)
    return jnp.where(gid < N, feats, 0.0)


def _fused_kernel(tbl_ref, p_ref, boxes_ref, scores_ref, classes_ref, sel_ref,
                  rows3_ref, cols3_ref, csupp_ref, rsupp_ref):
    s = pl.program_id(0)

    @pl.when(s < DSTEPS)
    def _decode():
        p = p_ref[...]
        for u in range(TA // TI):
            tile = s * (TA // TI) + u
            feats = _decode_tile(p[u * TI:(u + 1) * TI, :], tile * TI)
            rows3_ref[tile] = feats
            cols3_ref[tile] = feats.T

    @pl.when(s == DSTEPS)
    def _init():
        csupp_ref[...] = jnp.zeros((T, 8, TI), jnp.float32)

    @pl.when(s >= DSTEPS)
    def _suppress():
        t = tbl_ref[s, 0]
        q = tbl_ref[s, 1]
        first = tbl_ref[s, 2]
        last = tbl_ref[s, 3]

        @pl.when(first == 1)
        def _reset():
            rsupp_ref[...] = jnp.zeros((TI, 1), jnp.float32)

        r = rows3_ref[t]
        xi1 = r[:, 0:1]
        yi1 = r[:, 1:2]
        xi2 = r[:, 2:3]
        yi2 = r[:, 3:4]
        si = r[:, 4:5]
        ai = r[:, 6:7]

        def ov_block(c):
            ix1 = jnp.maximum(xi1, c[0:1, :])
            iy1 = jnp.maximum(yi1, c[1:2, :])
            ix2 = jnp.minimum(xi2, c[2:3, :])
            iy2 = jnp.minimum(yi2, c[3:4, :])
            iw = jnp.maximum(ix2 - ix1, 0.0)
            ih = jnp.maximum(iy2 - iy1, 0.0)
            inter = iw * ih
            union = (ai + c[6:7, :]) - inter
            iou = inter / jnp.maximum(union, 1e-9)
            return iou > IOU_THR

        ones_col = jnp.ones((TI, 1), jnp.float32)
        ones_row = jnp.ones((1, TI), jnp.float32)

        for u in range(CW):
            jt = q * CW + u

            @pl.when(jt < t)
            def _offdiag(jt=jt):
                c = cols3_ref[jt]
                ov = ov_block(c)
                hi = c[4:5, :] >= si
                ov_f = ov.astype(jnp.float32)
                row_f = (hi & ov).astype(jnp.float32)
                rowcnt = jnp.dot(row_f, ones_col,
                                 preferred_element_type=jnp.float32)
                rsupp_ref[...] = rsupp_ref[...] + rowcnt
                colcnt = jnp.dot(ones_row, ov_f - row_f,
                                 preferred_element_type=jnp.float32)
                csupp_ref[jt, 0:1, :] = csupp_ref[jt, 0:1, :] + colcnt

            @pl.when(jt == t)
            def _diag(jt=jt):
                c = cols3_ref[jt]
                ov = ov_block(c)
                sj = c[4:5, :]
                ii = t * TI + jax.lax.broadcasted_iota(jnp.int32, (TI, 1), 0)
                jj = t * TI + jax.lax.broadcasted_iota(jnp.int32, (1, TI), 1)
                hi = (sj > si) | ((sj == si) & (jj < ii))
                row_f = (hi & ov).astype(jnp.float32)
                rowcnt = jnp.dot(row_f, ones_col,
                                 preferred_element_type=jnp.float32)
                rsupp_ref[...] = rsupp_ref[...] + rowcnt

        @pl.when(last == 1)
        def _finalize():
            cs = csupp_ref[t].T
            supp = rsupp_ref[...] + cs[:, 0:1]
            keep = supp == 0.0
            kf = keep.astype(jnp.float32)
            ii = t * TI + jax.lax.broadcasted_iota(jnp.int32, (TI, 1), 0)
            boxes_ref[t] = r[:, 0:4] * kf
            scores_ref[t] = r[:, 4:5] * kf
            classes_ref[t] = jnp.where(keep, r[:, 5:6], 0.0).astype(jnp.int32)
            sel_ref[t] = jnp.where(keep, ii, -1)


def kernel(y_pred):
    p = jnp.reshape(y_pred, (N, PRED))
    p = jnp.pad(p, ((0, NPAD - N), (0, 0)))
    tbl = jnp.asarray(_tbl, dtype=jnp.int32)
    boxes, scores, classes, selected = pl.pallas_call(
        _fused_kernel,
        grid=(GRID,),
        in_specs=[
            pl.BlockSpec(memory_space=pltpu.SMEM),
            pl.BlockSpec((TA, PRED), lambda s: (jnp.minimum(s, DSTEPS - 1), 0)),
        ],
        out_specs=[
            pl.BlockSpec((T, TI, 4), lambda s: (0, 0, 0)),
            pl.BlockSpec((T, TI, 1), lambda s: (0, 0, 0)),
            pl.BlockSpec((T, TI, 1), lambda s: (0, 0, 0)),
            pl.BlockSpec((T, TI, 1), lambda s: (0, 0, 0)),
        ],
        out_shape=[
            jax.ShapeDtypeStruct((T, TI, 4), jnp.float32),
            jax.ShapeDtypeStruct((T, TI, 1), jnp.float32),
            jax.ShapeDtypeStruct((T, TI, 1), jnp.int32),
            jax.ShapeDtypeStruct((T, TI, 1), jnp.int32),
        ],
        scratch_shapes=[
            pltpu.VMEM((T, TI, 8), jnp.float32),
            pltpu.VMEM((T, 8, TI), jnp.float32),
            pltpu.VMEM((T, 8, TI), jnp.float32),
            pltpu.VMEM((TI, 1), jnp.float32),
        ],
        compiler_params=pltpu.CompilerParams(
            dimension_semantics=("arbitrary",),
        ),
    )(tbl, p)
    boxes = boxes.reshape(NPAD, 4)[:N]
    scores = scores.reshape(NPAD)[:N]
    classes = classes.reshape(NPAD)[:N]
    selected = selected.reshape(NPAD)[:N]
    return (boxes, scores, classes, selected)


# speedup vs baseline: 1.0016x; 1.0016x over previous
import math

import jax
import jax.numpy as jnp
from jax.experimental import pallas as pl
from jax.experimental.pallas import tpu as pltpu

N = 5000
PRED = 85
NCLS = 80
NPAD = 5120
TI = 1024
T = NPAD // TI
TA = 1024
DSTEPS = NPAD // TA
CW = 2
IOU_THR = 0.5

_steps = []
for _t in range(T - 1, -1, -1):
    _nq = math.ceil((_t + 1) / CW)
    for _q in range(_nq):
        _steps.append((_t, _q, 1 if _q == 0 else 0, 1 if _q == _nq - 1 else 0))
SSTEPS = len(_steps)
GRID = DSTEPS + SSTEPS
_tbl = [[0, 0, 0, 0]] * DSTEPS + [list(s) for s in _steps]


def _decode_tile(p, base):
    cx = p[:, 0:1]
    cy = p[:, 1:2]
    w = p[:, 2:3]
    h = p[:, 3:4]
    conf = p[:, 4:5]
    cls = p[:, 5:PRED]
    m = jnp.max(cls, axis=1, keepdims=True)
    iota = jax.lax.broadcasted_iota(jnp.int32, cls.shape, 1)
    amax = jnp.min(jnp.where(cls == m, iota, NCLS), axis=1, keepdims=True)
    x1 = cx - w * 0.5
    y1 = cy - h * 0.5
    x2 = cx + w * 0.5
    y2 = cy + h * 0.5
    score = conf * m
    area = jnp.maximum(x2 - x1, 0.0) * jnp.maximum(y2 - y1, 0.0)
    zero = jnp.zeros_like(score)
    feats = jnp.concatenate(
        [x1, y1, x2, y2, score, amax.astype(jnp.float32), area, zero], axis=1
    )
    gid = base + jax.lax.broadcasted_iota(jnp.int32, (TI, 1), 0)
    return jnp.where(gid < N, feats, 0.0)


def _fused_kernel(tbl_ref, p_ref, boxes_ref, scores_ref, classes_ref, sel_ref,
                  rows3_ref, cols3_ref, csupp_ref, rsupp_ref):
    s = pl.program_id(0)

    @pl.when(s < DSTEPS)
    def _decode():
        p = p_ref[...]
        for u in range(TA // TI):
            tile = s * (TA // TI) + u
            feats = _decode_tile(p[u * TI:(u + 1) * TI, :], tile * TI)
            rows3_ref[tile] = feats
            cols3_ref[tile] = feats.T

    @pl.when(s == DSTEPS)
    def _init():
        csupp_ref[...] = jnp.zeros((T, 8, TI), jnp.float32)

    @pl.when(s >= DSTEPS)
    def _suppress():
        t = tbl_ref[s, 0]
        q = tbl_ref[s, 1]
        first = tbl_ref[s, 2]
        last = tbl_ref[s, 3]

        @pl.when(first == 1)
        def _reset():
            rsupp_ref[...] = jnp.zeros((TI, 1), jnp.float32)

        r = rows3_ref[t]
        xi1 = r[:, 0:1]
        yi1 = r[:, 1:2]
        xi2 = r[:, 2:3]
        yi2 = r[:, 3:4]
        si = r[:, 4:5]
        ai = r[:, 6:7]

        def ov_block(c):
            ix1 = jnp.maximum(xi1, c[0:1, :])
            iy1 = jnp.maximum(yi1, c[1:2, :])
            ix2 = jnp.minimum(xi2, c[2:3, :])
            iy2 = jnp.minimum(yi2, c[3:4, :])
            iw = jnp.maximum(ix2 - ix1, 0.0)
            ih = jnp.maximum(iy2 - iy1, 0.0)
            inter = iw * ih
            union = (ai + c[6:7, :]) - inter
            iou = inter / jnp.maximum(union, 1e-9)
            return iou > IOU_THR

        ones_col = jnp.ones((TI, 1), jnp.float32)
        ones_row = jnp.ones((1, TI), jnp.float32)

        for u in range(CW):
            jt = q * CW + u

            @pl.when(jt < t)
            def _offdiag(jt=jt):
                c = cols3_ref[jt]
                ov = ov_block(c)
                hi = c[4:5, :] >= si
                ov_f = ov.astype(jnp.float32)
                row_f = (hi & ov).astype(jnp.float32)
                rowcnt = jnp.dot(row_f, ones_col,
                                 preferred_element_type=jnp.float32)
                rsupp_ref[...] = rsupp_ref[...] + rowcnt
                colcnt = jnp.dot(ones_row, ov_f - row_f,
                                 preferred_element_type=jnp.float32)
                csupp_ref[jt, 0:1, :] = csupp_ref[jt, 0:1, :] + colcnt

            @pl.when(jt == t)
            def _diag(jt=jt):
                c = cols3_ref[jt]
                ov = ov_block(c)
                sj = c[4:5, :]
                ii = t * TI + jax.lax.broadcasted_iota(jnp.int32, (TI, 1), 0)
                jj = t * TI + jax.lax.broadcasted_iota(jnp.int32, (1, TI), 1)
                hi = (sj > si) | ((sj == si) & (jj < ii))
                row_f = (hi & ov).astype(jnp.float32)
                rowcnt = jnp.dot(row_f, ones_col,
                                 preferred_element_type=jnp.float32)
                rsupp_ref[...] = rsupp_ref[...] + rowcnt

        @pl.when(last == 1)
        def _finalize():
            cs = csupp_ref[t].T
            supp = rsupp_ref[...] + cs[:, 0:1]
            keep = supp == 0.0
            kf = keep.astype(jnp.float32)
            ii = t * TI + jax.lax.broadcasted_iota(jnp.int32, (TI, 1), 0)
            boxes_ref[t] = r[:, 0:4] * kf
            scores_ref[t] = r[:, 4:5] * kf
            classes_ref[t] = jnp.where(keep, r[:, 5:6], 0.0).astype(jnp.int32)
            sel_ref[t] = jnp.where(keep, ii, -1)


def kernel(y_pred):
    p = jnp.reshape(y_pred, (N, PRED))
    p = jnp.pad(p, ((0, NPAD - N), (0, 0)))
    tbl = jnp.asarray(_tbl, dtype=jnp.int32)
    boxes, scores, classes, selected = pl.pallas_call(
        _fused_kernel,
        grid=(GRID,),
        in_specs=[
            pl.BlockSpec(memory_space=pltpu.SMEM),
            pl.BlockSpec((TA, PRED), lambda s: (jnp.minimum(s, DSTEPS - 1), 0)),
        ],
        out_specs=[
            pl.BlockSpec((T, TI, 4), lambda s: (0, 0, 0)),
            pl.BlockSpec((T, TI, 1), lambda s: (0, 0, 0)),
            pl.BlockSpec((T, TI, 1), lambda s: (0, 0, 0)),
            pl.BlockSpec((T, TI, 1), lambda s: (0, 0, 0)),
        ],
        out_shape=[
            jax.ShapeDtypeStruct((T, TI, 4), jnp.float32),
            jax.ShapeDtypeStruct((T, TI, 1), jnp.float32),
            jax.ShapeDtypeStruct((T, TI, 1), jnp.int32),
            jax.ShapeDtypeStruct((T, TI, 1), jnp.int32),
        ],
        scratch_shapes=[
            pltpu.VMEM((T, TI, 8), jnp.float32),
            pltpu.VMEM((T, 8, TI), jnp.float32),
            pltpu.VMEM((T, 8, TI), jnp.float32),
            pltpu.VMEM((TI, 1), jnp.float32),
        ],
        compiler_params=pltpu.CompilerParams(
            dimension_semantics=("arbitrary",),
        ),
    )(tbl, p)
    boxes = boxes.reshape(NPAD, 4)[:N]
    scores = scores.reshape(NPAD)[:N]
    classes = classes.reshape(NPAD)[:N]
    selected = selected.reshape(NPAD)[:N]
    return (boxes, scores, classes, selected)
